# Initial kernel scaffold; baseline (speedup 1.0000x reference)
import jax
import jax.numpy as jnp
from jax.experimental import pallas as pl


def kernel(x, w_gate, b_gate, w_fc, w_proj):
    raise NotImplementedError("write your pallas kernel here")


# speedup vs baseline: 1.9467x; 1.9467x over previous
import functools

import jax
import jax.numpy as jnp
from jax.experimental import pallas as pl
from jax.experimental.pallas import tpu as pltpu


def _moe_kernel(x_ref, wg_ref, bg_ref, wfc_ref, wproj_ref, out_ref, gate_ref,
                *, block_w, num_experts, top_k):
    i = pl.program_id(0)

    @pl.when(i == 0)
    def _router():
        x = x_ref[...]
        logits = jnp.dot(x, wg_ref[...], preferred_element_type=jnp.float32,
                         precision=jax.lax.Precision.HIGHEST) + bg_ref[...]
        m = jnp.max(logits, axis=-1, keepdims=True)
        p = jnp.exp(logits - m)
        p = p / jnp.sum(p, axis=-1, keepdims=True)
        eidx = jax.lax.broadcasted_iota(jnp.int32, p.shape, 1)
        rank = jnp.zeros(p.shape, dtype=jnp.int32)
        for j in range(num_experts):
            pj = p[:, j:j + 1]
            beats = (pj > p) | ((pj == p) & (j < eidx))
            rank = rank + beats.astype(jnp.int32)
        sel = (rank < top_k).astype(jnp.float32)
        psel = p * sel
        denom = jnp.sum(psel, axis=-1, keepdims=True)
        gate_ref[...] = psel / denom
        out_ref[...] = jnp.zeros_like(out_ref)

    x = x_ref[...]
    h = jnp.dot(x, wfc_ref[...], preferred_element_type=jnp.float32,
                precision=jax.lax.Precision.HIGHEST)
    h = jax.nn.gelu(h, approximate=True)
    gate = gate_ref[...]
    col_mod = jax.lax.broadcasted_iota(
        jnp.int32, (h.shape[0], block_w), 1) % num_experts
    scale = jnp.zeros_like(h)
    for e in range(num_experts):
        scale = scale + jnp.where(col_mod == e, gate[:, e:e + 1], 0.0)
    h = h * scale
    out_ref[...] += jnp.dot(h, wproj_ref[...],
                            preferred_element_type=jnp.float32,
                            precision=jax.lax.Precision.HIGHEST)


def kernel(x, w_gate, b_gate, w_fc, w_proj):
    B, T, C = x.shape
    E = w_gate.shape[-1]
    HE = w_fc.shape[-1]
    H = HE // E
    x2 = x.reshape(B * T, C)
    bg2 = b_gate.reshape(1, E)
    wp_r = w_proj.reshape(H, C, E).transpose(0, 2, 1).reshape(HE, C)

    block_w = 1536
    num_blocks = HE // block_w

    out = pl.pallas_call(
        functools.partial(_moe_kernel, block_w=block_w, num_experts=E,
                          top_k=2),
        grid=(num_blocks,),
        in_specs=[
            pl.BlockSpec((B * T, C), lambda i: (0, 0)),
            pl.BlockSpec((C, E), lambda i: (0, 0)),
            pl.BlockSpec((1, E), lambda i: (0, 0)),
            pl.BlockSpec((C, block_w), lambda i: (0, i)),
            pl.BlockSpec((block_w, C), lambda i: (i, 0)),
        ],
        out_specs=pl.BlockSpec((B * T, C), lambda i: (0, 0)),
        out_shape=jax.ShapeDtypeStruct((B * T, C), jnp.float32),
        scratch_shapes=[pltpu.VMEM((B * T, E), jnp.float32)],
        compiler_params=pltpu.CompilerParams(
            dimension_semantics=("arbitrary",)),
    )(x2, w_gate, bg2, w_fc, wp_r)
    return out.reshape(B, T, C)


# speedup vs baseline: 2.0385x; 1.0471x over previous
import functools

import jax
import jax.numpy as jnp
from jax.experimental import pallas as pl
from jax.experimental.pallas import tpu as pltpu


def _moe_kernel(x_ref, wg_ref, bg_ref, wfc_ref, wproj_ref, out_ref, gate_ref,
                *, block_w, num_experts, top_k):
    i = pl.program_id(0)

    @pl.when(i == 0)
    def _router():
        x = x_ref[...]
        logits = jnp.dot(x, wg_ref[...], preferred_element_type=jnp.float32,
                         precision=jax.lax.Precision.HIGHEST) + bg_ref[...]
        m = jnp.max(logits, axis=-1, keepdims=True)
        p = jnp.exp(logits - m)
        p = p / jnp.sum(p, axis=-1, keepdims=True)
        eidx = jax.lax.broadcasted_iota(jnp.int32, p.shape, 1)
        rank = jnp.zeros(p.shape, dtype=jnp.int32)
        for j in range(num_experts):
            pj = p[:, j:j + 1]
            beats = (pj > p) | ((pj == p) & (j < eidx))
            rank = rank + beats.astype(jnp.int32)
        sel = (rank < top_k).astype(jnp.float32)
        psel = p * sel
        denom = jnp.sum(psel, axis=-1, keepdims=True)
        gate_ref[...] = psel / denom
        out_ref[...] = jnp.zeros_like(out_ref)

    x = x_ref[...]
    h = jnp.dot(x, wfc_ref[...], preferred_element_type=jnp.float32)
    h = jax.nn.gelu(h, approximate=True)
    gate = gate_ref[...]
    col_mod = jax.lax.broadcasted_iota(
        jnp.int32, (h.shape[0], block_w), 1) % num_experts
    scale = jnp.zeros_like(h)
    for e in range(num_experts):
        scale = scale + jnp.where(col_mod == e, gate[:, e:e + 1], 0.0)
    h = h * scale
    out_ref[...] += jnp.dot(h, wproj_ref[...],
                            preferred_element_type=jnp.float32)


def kernel(x, w_gate, b_gate, w_fc, w_proj):
    B, T, C = x.shape
    E = w_gate.shape[-1]
    HE = w_fc.shape[-1]
    H = HE // E
    x2 = x.reshape(B * T, C)
    bg2 = b_gate.reshape(1, E)
    wp_r = w_proj.reshape(H, C, E).transpose(0, 2, 1).reshape(HE, C)

    block_w = 1536
    num_blocks = HE // block_w

    out = pl.pallas_call(
        functools.partial(_moe_kernel, block_w=block_w, num_experts=E,
                          top_k=2),
        grid=(num_blocks,),
        in_specs=[
            pl.BlockSpec((B * T, C), lambda i: (0, 0)),
            pl.BlockSpec((C, E), lambda i: (0, 0)),
            pl.BlockSpec((1, E), lambda i: (0, 0)),
            pl.BlockSpec((C, block_w), lambda i: (0, i)),
            pl.BlockSpec((block_w, C), lambda i: (i, 0)),
        ],
        out_specs=pl.BlockSpec((B * T, C), lambda i: (0, 0)),
        out_shape=jax.ShapeDtypeStruct((B * T, C), jnp.float32),
        scratch_shapes=[pltpu.VMEM((B * T, E), jnp.float32)],
        compiler_params=pltpu.CompilerParams(
            dimension_semantics=("arbitrary",)),
    )(x2, w_gate, bg2, w_fc, wp_r)
    return out.reshape(B, T, C)
